# Initial kernel scaffold; baseline (speedup 1.0000x reference)
import jax
import jax.numpy as jnp
from jax.experimental import pallas as pl


def kernel(inputs, embedding_weight):
    raise NotImplementedError("write your pallas kernel here")


# speedup vs baseline: 1.5049x; 1.5049x over previous
import functools

import jax
import jax.numpy as jnp
from jax.experimental import pallas as pl
from jax.experimental.pallas import tpu as pltpu
from jax.experimental.pallas import tpu_sc as plsc

N_EMB = 8192
D_EMB = 256
N_IMG = 16
P_IMG = 1024
B_TOT = N_IMG * P_IMG
CB_CHUNK = 1024
COMMIT = 0.25


def _vq_argmin_body(x_ref, e_ref, idx_ref, loss_ref, acc_ref):
    n = pl.program_id(0)
    x = x_ref[0]
    x2 = jnp.sum(x * x, axis=0, keepdims=True)
    best_v = None
    best_i = None
    for k in range(N_EMB // CB_CHUNK):
        e = e_ref[k * CB_CHUNK:(k + 1) * CB_CHUNK, :]
        e2 = jnp.sum(e * e, axis=1, keepdims=True)
        mm = jax.lax.dot_general(
            e, x, (((1,), (0,)), ((), ())),
            preferred_element_type=jnp.float32)
        dist = (x2 + e2) - 2.0 * mm
        lv = jnp.min(dist, axis=0, keepdims=True)
        ci = jax.lax.broadcasted_iota(jnp.int32, (CB_CHUNK, P_IMG), 0)
        li = jnp.min(jnp.where(dist == lv, ci, CB_CHUNK),
                     axis=0, keepdims=True) + k * CB_CHUNK
        if best_v is None:
            best_v, best_i = lv, li
        else:
            upd = lv < best_v
            best_v = jnp.where(upd, lv, best_v)
            best_i = jnp.where(upd, li, best_i)
    idx_ref[0] = best_i

    @pl.when(n == 0)
    def _init():
        acc_ref[0] = 0.0

    acc_ref[0] += jnp.sum(best_v)

    @pl.when(n == N_IMG - 1)
    def _fin():
        val = acc_ref[0] * ((1.0 + COMMIT) / (B_TOT * D_EMB))
        loss_ref[...] = jnp.full((1, 1), val, dtype=jnp.float32)


_vq_argmin = pl.pallas_call(
    _vq_argmin_body,
    grid=(N_IMG,),
    in_specs=[
        pl.BlockSpec((1, D_EMB, P_IMG), lambda n: (n, 0, 0)),
        pl.BlockSpec((N_EMB, D_EMB), lambda n: (0, 0)),
    ],
    out_specs=[
        pl.BlockSpec((1, 1, P_IMG), lambda n: (n, 0, 0)),
        pl.BlockSpec((1, 1), lambda n: (0, 0)),
    ],
    out_shape=[
        jax.ShapeDtypeStruct((N_IMG, 1, P_IMG), jnp.int32),
        jax.ShapeDtypeStruct((1, 1), jnp.float32),
    ],
    scratch_shapes=[pltpu.SMEM((1,), jnp.float32)],
)


_SC_NC = 2
_SC_NS = 16
_NW = _SC_NC * _SC_NS
_BPW = B_TOT // _NW
_CH = 256

@functools.cache
def _gather_rows_kernel():
    mesh = plsc.VectorSubcoreMesh(core_axis_name="c", subcore_axis_name="s")

    @functools.partial(
        pl.kernel,
        mesh=mesh,
        out_type=jax.ShapeDtypeStruct((B_TOT, D_EMB), jnp.float32),
        scratch_types=[
            pltpu.VMEM((_CH,), jnp.int32),
            pltpu.VMEM((_CH, D_EMB), jnp.float32),
            pltpu.SemaphoreType.DMA,
        ],
    )
    def _gather_rows(table_hbm, idx_hbm, out_hbm, idx_v, rows_v, sem):
        wid = jax.lax.axis_index("s") * _SC_NC + jax.lax.axis_index("c")
        base = wid * _BPW
        for t in range(_BPW // _CH):
            off = base + t * _CH
            pltpu.sync_copy(idx_hbm.at[pl.ds(off, _CH)], idx_v)
            pltpu.async_copy(table_hbm.at[idx_v], rows_v, sem).wait()
            pltpu.sync_copy(rows_v, out_hbm.at[pl.ds(off, _CH)])

    return _gather_rows


def kernel(inputs, embedding_weight):
    x3 = inputs.reshape(N_IMG, D_EMB, P_IMG)
    idx3, loss11 = _vq_argmin(x3, embedding_weight)
    idx = idx3.reshape(B_TOT)
    q = _gather_rows_kernel()(embedding_weight, idx)
    out = q.reshape(N_IMG, 32, 32, D_EMB).transpose(0, 3, 1, 2)
    return (out, loss11[0, 0])


# speedup vs baseline: 1.5528x; 1.0318x over previous
import functools

import jax
import jax.numpy as jnp
from jax.experimental import pallas as pl
from jax.experimental.pallas import tpu as pltpu
from jax.experimental.pallas import tpu_sc as plsc

N_EMB = 8192
D_EMB = 256
N_IMG = 16
P_IMG = 1024
B_TOT = N_IMG * P_IMG
CB_CHUNK = 1024
COMMIT = 0.25


def _vq_argmin_body(x_ref, e_ref, idx_ref, loss_ref, acc_ref, e2_ref):
    n = pl.program_id(0)

    @pl.when(n == 0)
    def _pre():
        for k in range(N_EMB // CB_CHUNK):
            e = e_ref[k * CB_CHUNK:(k + 1) * CB_CHUNK, :]
            e2_ref[k * CB_CHUNK:(k + 1) * CB_CHUNK, :] = (
                jnp.sum(e * e, axis=1, keepdims=True))

    x = x_ref[0]
    x2 = jnp.sum(x * x, axis=0, keepdims=True)
    best_v = None
    best_i = None
    for k in range(N_EMB // CB_CHUNK):
        e = e_ref[k * CB_CHUNK:(k + 1) * CB_CHUNK, :]
        e2 = e2_ref[k * CB_CHUNK:(k + 1) * CB_CHUNK, :]
        mm = jax.lax.dot_general(
            e, x, (((1,), (0,)), ((), ())),
            preferred_element_type=jnp.float32)
        score = e2 - 2.0 * mm
        lv = jnp.min(score, axis=0, keepdims=True)
        ci = jax.lax.broadcasted_iota(jnp.int32, (CB_CHUNK, P_IMG), 0)
        li = jnp.min(jnp.where(score == lv, ci, CB_CHUNK),
                     axis=0, keepdims=True) + k * CB_CHUNK
        if best_v is None:
            best_v, best_i = lv, li
        else:
            upd = lv < best_v
            best_v = jnp.where(upd, lv, best_v)
            best_i = jnp.where(upd, li, best_i)
    best_v = x2 + best_v
    idx_ref[0] = best_i

    @pl.when(n == 0)
    def _init():
        acc_ref[0] = 0.0

    acc_ref[0] += jnp.sum(best_v)

    @pl.when(n == N_IMG - 1)
    def _fin():
        val = acc_ref[0] * ((1.0 + COMMIT) / (B_TOT * D_EMB))
        loss_ref[...] = jnp.full((1, 1), val, dtype=jnp.float32)


_vq_argmin = pl.pallas_call(
    _vq_argmin_body,
    grid=(N_IMG,),
    in_specs=[
        pl.BlockSpec((1, D_EMB, P_IMG), lambda n: (n, 0, 0)),
        pl.BlockSpec((N_EMB, D_EMB), lambda n: (0, 0)),
    ],
    out_specs=[
        pl.BlockSpec((1, 1, P_IMG), lambda n: (n, 0, 0)),
        pl.BlockSpec((1, 1), lambda n: (0, 0)),
    ],
    out_shape=[
        jax.ShapeDtypeStruct((N_IMG, 1, P_IMG), jnp.int32),
        jax.ShapeDtypeStruct((1, 1), jnp.float32),
    ],
    scratch_shapes=[
        pltpu.SMEM((1,), jnp.float32),
        pltpu.VMEM((N_EMB, 1), jnp.float32),
    ],
)


_SC_NC = 2
_SC_NS = 16
_NW = _SC_NC * _SC_NS
_BPW = B_TOT // _NW
_CH = 256

@functools.cache
def _gather_rows_kernel():
    mesh = plsc.VectorSubcoreMesh(core_axis_name="c", subcore_axis_name="s")

    @functools.partial(
        pl.kernel,
        mesh=mesh,
        out_type=jax.ShapeDtypeStruct((B_TOT, D_EMB), jnp.float32),
        scratch_types=[
            pltpu.VMEM((_CH,), jnp.int32),
            pltpu.VMEM((_CH, D_EMB), jnp.float32),
            pltpu.SemaphoreType.DMA,
        ],
    )
    def _gather_rows(table_hbm, idx_hbm, out_hbm, idx_v, rows_v, sem):
        wid = jax.lax.axis_index("s") * _SC_NC + jax.lax.axis_index("c")
        base = wid * _BPW
        for t in range(_BPW // _CH):
            off = base + t * _CH
            pltpu.sync_copy(idx_hbm.at[pl.ds(off, _CH)], idx_v)
            pltpu.async_copy(table_hbm.at[idx_v], rows_v, sem).wait()
            pltpu.sync_copy(rows_v, out_hbm.at[pl.ds(off, _CH)])

    return _gather_rows


def kernel(inputs, embedding_weight):
    x3 = inputs.reshape(N_IMG, D_EMB, P_IMG)
    idx3, loss11 = _vq_argmin(x3, embedding_weight)
    idx = idx3.reshape(B_TOT)
    q = _gather_rows_kernel()(embedding_weight, idx)
    out = q.reshape(N_IMG, 32, 32, D_EMB).transpose(0, 3, 1, 2)
    return (out, loss11[0, 0])
